# baseline (speedup 1.0000x reference)
import functools

import jax
import jax.numpy as jnp
from jax import lax
from jax.experimental import pallas as pl
from jax.experimental.pallas import tpu as pltpu
from jax.experimental.pallas import tpu_sc as plsc

H = 8
D = 32
C = H * D
N = 10000
E = 160000
NODE_TYPES = ["author", "paper"]
EDGE_TYPES = [("author", "writes", "paper"), ("paper", "rev_writes", "author"), ("paper", "cites", "paper")]

NSUB = 16
WIN = 128
NWIN = 79
WINB = 64
NWINB = 2 * NWIN
ET = WIN * NWIN
EPAD = ET * NSUB - E
NP = N + 16
SEGP = 632 * NSUB
HALF = SEGP // 2
SEGH = HALF + 8
SEG_ALL = 2560 * NSUB


def _proj_body(x_ref, w_ref, b_ref, o_ref):
    o_ref[...] = jnp.dot(x_ref[...], w_ref[...], preferred_element_type=jnp.float32) + b_ref[...]


def _proj(x, W, b, block_rows=1000):
    n, d_in = x.shape
    c = W.shape[1]
    return pl.pallas_call(
        _proj_body,
        grid=(n // block_rows,),
        in_specs=[
            pl.BlockSpec((block_rows, d_in), lambda i: (i, 0)),
            pl.BlockSpec((d_in, c), lambda i: (0, 0)),
            pl.BlockSpec((c,), lambda i: (0,)),
        ],
        out_specs=pl.BlockSpec((block_rows, c), lambda i: (i, 0)),
        out_shape=jax.ShapeDtypeStruct((n, c), jnp.float32),
    )(x, W, b)


def _sc_edge_body(src_ref, dst_ref, ats_ref, atd_ref, mrow_ref, x16_ref, out_ref,
                  src_v, dst_v, didx1, didx2, e4, asrc_v, adst_v, zrow, buf0, bufs, widx3, mr_v,
                  seg_all, acc_sh, sem, sems, ssem):
    t = lax.axis_index("s")
    c = lax.axis_index("c")
    zero16 = jnp.zeros((16,), jnp.float32)

    pltpu.sync_copy(src_ref.at[t], src_v)
    pltpu.sync_copy(dst_ref.at[t], dst_v)

    def _ridx(j, carry):
        for q in range(8):
            sl = pl.ds(q * 16, 16)
            jv = dst_v[j, sl]
            lo = jv < HALF
            didx1[j, sl] = jnp.where(lo, jv, HALF)
            didx2[j, sl] = jnp.where(lo, HALF, jv - HALF)
        return carry
    lax.fori_loop(0, NWIN, _ridx, 0)

    def _zrow(i, carry):
        zrow[pl.ds(i * 16, 16)] = zero16
        return carry
    lax.fori_loop(0, 40, _zrow, 0)

    def _zbuf(r, carry):
        buf0[r, pl.ds(0, 16)] = zero16
        return carry
    lax.fori_loop(0, WIN, _zbuf, 0)

    for ch in range(4):
        pltpu.sync_copy(zrow.at[pl.ds(0, 640)], seg_all.at[pl.ds(t * 2560 + ch * 640, 640)])
    zbase = t * 626
    for off, sz in ((0, 128), (128, 128), (256, 128), (384, 128), (512, 114)):
        pltpu.sync_copy(buf0.at[pl.ds(0, sz)], acc_sh.at[pl.ds(zbase + off, sz)])
    plsc.subcore_barrier()

    for h in range(4):
        hg = c * 4 + h
        pltpu.sync_copy(ats_ref.at[pl.ds(hg * N, N)], asrc_v.at[pl.ds(0, N)])
        pltpu.sync_copy(atd_ref.at[pl.ds(hg * N, N)], adst_v.at[pl.ds(0, N)])
        pltpu.sync_copy(mrow_ref.at[pl.ds(hg * 16, 16)], mr_v)
        mr = mr_v[...]

        def _pa(j, carry):
            for q in range(8):
                sl = pl.ds(q * 16, 16)
                iv = src_v[j, sl]
                jv = dst_v[j, sl]
                av = plsc.load_gather(asrc_v, [iv])
                bv = plsc.load_gather(adst_v, [jv])
                al = av + bv
                al = jnp.where(al >= 0.0, al, al * 0.2)
                ev = jnp.exp(al - mr)
                e4[h, pl.ds(j * 128 + q * 16, 16)] = ev
            return carry
        lax.fori_loop(0, NWIN, _pa, 0)

    def _sca(i, carry):
        b = i & 3

        @pl.when(i < NWIN * 4)
        def _():
            j = i >> 2
            h = i & 3
            w_sl = pl.ds(j * 128, 128)
            for q in range(8):
                sl = pl.ds(q * 16, 16)
                widx3[b, sl] = didx1[j, sl] + (2 * h) * SEGH
                widx3[4 + b, sl] = didx2[j, sl] + (2 * h + 1) * SEGH
            pltpu.async_copy(e4.at[h, w_sl], seg_all.at[widx3.at[b]], sems.at[b], add=True)
            pltpu.async_copy(e4.at[h, w_sl], seg_all.at[widx3.at[4 + b]], sems.at[4 + b], add=True)

        @pl.when(i >= 3)
        def _():
            im = i - 3
            bm = im & 3
            jm = im >> 2
            hm = im & 3
            w_slm = pl.ds(jm * 128, 128)
            pltpu.make_async_copy(e4.at[hm, w_slm], seg_all.at[widx3.at[bm]], sems.at[bm]).wait()
            pltpu.make_async_copy(e4.at[hm, w_slm], seg_all.at[widx3.at[4 + bm]], sems.at[4 + bm]).wait()
        return carry
    lax.fori_loop(0, NWIN * 4 + 3, _sca, 0)
    plsc.subcore_barrier()

    for h in range(4):
        pltpu.sync_copy(seg_all.at[pl.ds((2 * h) * SEGH, HALF)], asrc_v.at[pl.ds(0, HALF)])
        pltpu.sync_copy(seg_all.at[pl.ds((2 * h + 1) * SEGH, HALF)], asrc_v.at[pl.ds(HALF, HALF)])

        def _pa2(j, carry):
            for q in range(8):
                sl = pl.ds(j * 128 + q * 16, 16)
                jv = dst_v[j, pl.ds(q * 16, 16)]
                sv = plsc.load_gather(asrc_v, [jv])
                ev = e4[h, sl]
                e4[h, sl] = ev / (sv + 1e-16)
            return carry
        lax.fori_loop(0, NWIN, _pa2, 0)

    cb = t * 625
    for p in range(8):
        g_glob = c * 8 + p

        if p > 0:
            for off, sz in ((0, 128), (128, 128), (256, 128), (384, 128), (512, 114)):
                pltpu.sync_copy(buf0.at[pl.ds(0, sz)], acc_sh.at[pl.ds(zbase + off, sz)])
            plsc.subcore_barrier()

        def _pb(j, carry):
            b = j & 3

            @pl.when(jnp.logical_and(j >= 4, j - 4 < NWIN))
            def _():
                pltpu.make_async_copy(bufs.at[b], acc_sh.at[dst_v.at[j - 4]], ssem).wait()

            @pl.when(j < NWIN)
            def _():
                for q in range(8):
                    sl = pl.ds(q * 16, 16)
                    widx3[b, sl] = src_v[j, sl] * 16 + g_glob
                pltpu.async_copy(x16_ref.at[widx3.at[b]], bufs.at[b], sems.at[b])

            @pl.when(jnp.logical_and(j >= 3, j - 3 < NWIN))
            def _():
                jm = j - 3
                bm = jm & 3
                pltpu.make_async_copy(x16_ref.at[widx3.at[bm]], bufs.at[bm], sems.at[bm]).wait()

                def _rb(g, carry2):
                    wv = e4[p // 2, pl.ds(jm * 128 + g * 16, 16)]
                    for a in range(16):
                        wb = jnp.full((16,), wv[a], jnp.float32)
                        r = g * 16 + a
                        bufs[bm, r, pl.ds(0, 16)] = bufs[bm, r, pl.ds(0, 16)] * wb
                    return carry2
                lax.fori_loop(0, 8, _rb, 0)
                pltpu.async_copy(bufs.at[bm], acc_sh.at[dst_v.at[jm]], ssem, add=True)
            return carry
        lax.fori_loop(0, NWIN + 4, _pb, 0)
        plsc.subcore_barrier()

        pltpu.sync_copy(acc_sh.at[pl.ds(cb, 625)], out_ref.at[g_glob, pl.ds(cb, 625)])
        plsc.subcore_barrier()


def _sc_edge(src3, dst3, ats, atd, mrow, x16):
    mesh = plsc.VectorSubcoreMesh(core_axis_name="c", subcore_axis_name="s")
    fn = pl.kernel(
        _sc_edge_body,
        out_type=jax.ShapeDtypeStruct((16, N, 16), jnp.float32),
        mesh=mesh,
        compiler_params=pltpu.CompilerParams(needs_layout_passes=False, use_tc_tiling_on_sc=False),
        scratch_types=[
            pltpu.VMEM((NWIN, WIN), jnp.int32),
            pltpu.VMEM((NWIN, WIN), jnp.int32),
            pltpu.VMEM((NWIN, WIN), jnp.int32),
            pltpu.VMEM((NWIN, WIN), jnp.int32),
            pltpu.VMEM((4, ET), jnp.float32),
            pltpu.VMEM((SEGP,), jnp.float32),
            pltpu.VMEM((SEGP,), jnp.float32),
            pltpu.VMEM((640,), jnp.float32),
            pltpu.VMEM((WIN, 16), jnp.float32),
            pltpu.VMEM((4, WIN, 16), jnp.float32),
            pltpu.VMEM((8, WIN), jnp.int32),
            pltpu.VMEM((16,), jnp.float32),
            pltpu.VMEM_SHARED((SEG_ALL,), jnp.float32),
            pltpu.VMEM_SHARED((NP, 16), jnp.float32),
            pltpu.SemaphoreType.DMA,
            pltpu.SemaphoreType.DMA((8,)),
            pltpu.SemaphoreType.DMA,
        ],
    )
    return fn(src3, dst3, ats, atd, mrow, x16)


def _pad_edges(ei):
    src = jnp.concatenate([ei[0], jnp.zeros((EPAD,), ei.dtype)]).astype(jnp.int32)
    dst = jnp.concatenate([ei[1], jnp.full((EPAD,), N, ei.dtype)]).astype(jnp.int32)
    return src.reshape(NSUB, NWIN, WIN), dst.reshape(NSUB, NWIN, WIN)


def _han_conv(x_dict, epad_dict, p):
    xn = {}
    out_lists = {t: [] for t in NODE_TYPES}
    for t in NODE_TYPES:
        W, b = p["proj"][t]
        xn[t] = _proj(x_dict[t], W, b)
    for et in EDGE_TYPES:
        src_t, _, dst_t = et
        k = "__".join(et)
        src3, dst3 = epad_dict[et]
        xs = xn[src_t]
        xd = xn[dst_t]
        ats = (xs.reshape(N, H, D) * p["att_src"][k]).sum(-1).T
        atd = (xd.reshape(N, H, D) * p["att_dst"][k]).sum(-1).T
        m = jnp.max(ats, axis=1) + jnp.max(atd, axis=1)
        m = jnp.where(m >= 0.0, m, m * 0.2)
        mrow = jnp.broadcast_to(m[:, None], (H, 16)).reshape(H * 16)
        ats = ats.reshape(H * N)
        atd = atd.reshape(H * N)
        x16 = xs.reshape(16 * N, 16)
        out3 = _sc_edge(src3, dst3, ats, atd, mrow, x16)
        out_lists[dst_t].append(jax.nn.relu(out3.transpose(1, 0, 2).reshape(N, C)))
    new_x = {}
    for t in NODE_TYPES:
        stacked = jnp.stack(out_lists[t])
        ksem = jnp.tanh(stacked @ p["k_W"] + p["k_b"]).mean(axis=1)
        score = (p["q"] * ksem).sum(-1)
        attn = jax.nn.softmax(score, axis=0)
        new_x[t] = (attn[:, None, None] * stacked).sum(0)
    return new_x


def kernel(x_author, x_paper, edge_index_writes, edge_index_rev_writes, edge_index_cites, params):
    x_dict = {"author": x_author, "paper": x_paper}
    epad_dict = {
        EDGE_TYPES[0]: _pad_edges(edge_index_writes),
        EDGE_TYPES[1]: _pad_edges(edge_index_rev_writes),
        EDGE_TYPES[2]: _pad_edges(edge_index_cites),
    }
    for lp in params:
        x_dict = _han_conv(x_dict, epad_dict, lp)
    return (x_dict["author"], x_dict["paper"])


# speedup vs baseline: 1.0449x; 1.0449x over previous
import functools

import jax
import jax.numpy as jnp
from jax import lax
from jax.experimental import pallas as pl
from jax.experimental.pallas import tpu as pltpu
from jax.experimental.pallas import tpu_sc as plsc

H = 8
D = 32
C = H * D
N = 10000
E = 160000
NODE_TYPES = ["author", "paper"]
EDGE_TYPES = [("author", "writes", "paper"), ("paper", "rev_writes", "author"), ("paper", "cites", "paper")]

NSUB = 16
WIN = 128
NWIN = 79
WINB = 64
NWINB = 2 * NWIN
ET = WIN * NWIN
EPAD = ET * NSUB - E
NP = N + 16
SEGP = 632 * NSUB
HALF = SEGP // 2
SEGH = HALF + 8
SEG_ALL = 2560 * NSUB


def _proj_body(x_ref, w_ref, b_ref, o_ref):
    o_ref[...] = jnp.dot(x_ref[...], w_ref[...], preferred_element_type=jnp.float32) + b_ref[...]


def _proj(x, W, b, block_rows=1000):
    n, d_in = x.shape
    c = W.shape[1]
    return pl.pallas_call(
        _proj_body,
        grid=(n // block_rows,),
        in_specs=[
            pl.BlockSpec((block_rows, d_in), lambda i: (i, 0)),
            pl.BlockSpec((d_in, c), lambda i: (0, 0)),
            pl.BlockSpec((c,), lambda i: (0,)),
        ],
        out_specs=pl.BlockSpec((block_rows, c), lambda i: (i, 0)),
        out_shape=jax.ShapeDtypeStruct((n, c), jnp.float32),
    )(x, W, b)


def _sem_body_r2(s0_ref, s1_ref, kw_ref, kb_ref, q_ref, o_ref):
    s0 = jnp.maximum(s0_ref[...], 0.0)
    s1 = jnp.maximum(s1_ref[...], 0.0)
    kw = kw_ref[...]
    kb = kb_ref[...]
    qv = q_ref[...]
    t0 = jnp.tanh(jnp.dot(s0, kw, preferred_element_type=jnp.float32) + kb)
    t1 = jnp.tanh(jnp.dot(s1, kw, preferred_element_type=jnp.float32) + kb)
    sc0 = jnp.sum(qv[0] * t0.mean(axis=0))
    sc1 = jnp.sum(qv[0] * t1.mean(axis=0))
    m = jnp.maximum(sc0, sc1)
    e0 = jnp.exp(sc0 - m)
    e1 = jnp.exp(sc1 - m)
    a0 = e0 / (e0 + e1)
    a1 = e1 / (e0 + e1)
    o_ref[...] = a0 * s0 + a1 * s1


def _sem_body_r1(s0_ref, o_ref):
    o_ref[...] = jnp.maximum(s0_ref[...], 0.0)


def _semantic(stk, kw, kb, q):
    n, c = stk[0].shape
    if len(stk) == 1:
        return pl.pallas_call(
            _sem_body_r1,
            out_shape=jax.ShapeDtypeStruct((n, c), jnp.float32),
        )(stk[0])
    return pl.pallas_call(
        _sem_body_r2,
        out_shape=jax.ShapeDtypeStruct((n, c), jnp.float32),
    )(stk[0], stk[1], kw, kb, q)


def _sc_edge_body(src_ref, dst_ref, ats_ref, atd_ref, mrow_ref, x16_ref, out_ref,
                  src_v, dst_v, didx1, didx2, e4, asrc_v, adst_v, zrow, buf0, bufs, widx3, mr_v,
                  seg_all, acc_sh, sem, sems, ssem):
    t = lax.axis_index("s")
    c = lax.axis_index("c")
    zero16 = jnp.zeros((16,), jnp.float32)

    pltpu.sync_copy(src_ref.at[t], src_v)
    pltpu.sync_copy(dst_ref.at[t], dst_v)

    def _ridx(j, carry):
        for q in range(8):
            sl = pl.ds(q * 16, 16)
            jv = dst_v[j, sl]
            lo = jv < HALF
            didx1[j, sl] = jnp.where(lo, jv, HALF)
            didx2[j, sl] = jnp.where(lo, HALF, jv - HALF)
        return carry
    lax.fori_loop(0, NWIN, _ridx, 0)

    def _zrow(i, carry):
        zrow[pl.ds(i * 16, 16)] = zero16
        return carry
    lax.fori_loop(0, 40, _zrow, 0)

    def _zbuf(r, carry):
        buf0[r, pl.ds(0, 16)] = zero16
        return carry
    lax.fori_loop(0, WIN, _zbuf, 0)

    for ch in range(4):
        pltpu.sync_copy(zrow.at[pl.ds(0, 640)], seg_all.at[pl.ds(t * 2560 + ch * 640, 640)])
    zbase = t * 626
    for off, sz in ((0, 128), (128, 128), (256, 128), (384, 128), (512, 114)):
        pltpu.sync_copy(buf0.at[pl.ds(0, sz)], acc_sh.at[pl.ds(zbase + off, sz)])
    plsc.subcore_barrier()

    for h in range(4):
        hg = c * 4 + h
        pltpu.sync_copy(ats_ref.at[pl.ds(hg * N, N)], asrc_v.at[pl.ds(0, N)])
        pltpu.sync_copy(atd_ref.at[pl.ds(hg * N, N)], adst_v.at[pl.ds(0, N)])
        pltpu.sync_copy(mrow_ref.at[pl.ds(hg * 16, 16)], mr_v)
        mr = mr_v[...]

        def _pa(j, carry):
            for q in range(8):
                sl = pl.ds(q * 16, 16)
                iv = src_v[j, sl]
                jv = dst_v[j, sl]
                av = plsc.load_gather(asrc_v, [iv])
                bv = plsc.load_gather(adst_v, [jv])
                al = av + bv
                al = jnp.where(al >= 0.0, al, al * 0.2)
                ev = jnp.exp(al - mr)
                e4[h, pl.ds(j * 128 + q * 16, 16)] = ev
            return carry
        lax.fori_loop(0, NWIN, _pa, 0)

    def _sca(i, carry):
        b = i & 3

        @pl.when(i < NWIN * 4)
        def _():
            j = i >> 2
            h = i & 3
            w_sl = pl.ds(j * 128, 128)
            for q in range(8):
                sl = pl.ds(q * 16, 16)
                widx3[b, sl] = didx1[j, sl] + (2 * h) * SEGH
                widx3[4 + b, sl] = didx2[j, sl] + (2 * h + 1) * SEGH
            pltpu.async_copy(e4.at[h, w_sl], seg_all.at[widx3.at[b]], sems.at[b], add=True)
            pltpu.async_copy(e4.at[h, w_sl], seg_all.at[widx3.at[4 + b]], sems.at[4 + b], add=True)

        @pl.when(i >= 3)
        def _():
            im = i - 3
            bm = im & 3
            jm = im >> 2
            hm = im & 3
            w_slm = pl.ds(jm * 128, 128)
            pltpu.make_async_copy(e4.at[hm, w_slm], seg_all.at[widx3.at[bm]], sems.at[bm]).wait()
            pltpu.make_async_copy(e4.at[hm, w_slm], seg_all.at[widx3.at[4 + bm]], sems.at[4 + bm]).wait()
        return carry
    lax.fori_loop(0, NWIN * 4 + 3, _sca, 0)
    plsc.subcore_barrier()

    for h in range(4):
        pltpu.sync_copy(seg_all.at[pl.ds((2 * h) * SEGH, HALF)], asrc_v.at[pl.ds(0, HALF)])
        pltpu.sync_copy(seg_all.at[pl.ds((2 * h + 1) * SEGH, HALF)], asrc_v.at[pl.ds(HALF, HALF)])

        def _pa2(j, carry):
            for q in range(8):
                sl = pl.ds(j * 128 + q * 16, 16)
                jv = dst_v[j, pl.ds(q * 16, 16)]
                sv = plsc.load_gather(asrc_v, [jv])
                ev = e4[h, sl]
                e4[h, sl] = ev / (sv + 1e-16)
            return carry
        lax.fori_loop(0, NWIN, _pa2, 0)

    cb = t * 625
    for p in range(8):
        g_glob = c * 8 + p

        if p > 0:
            for off, sz in ((0, 128), (128, 128), (256, 128), (384, 128), (512, 114)):
                pltpu.sync_copy(buf0.at[pl.ds(0, sz)], acc_sh.at[pl.ds(zbase + off, sz)])
            plsc.subcore_barrier()

        def _pb(j, carry):
            b = j & 3

            @pl.when(jnp.logical_and(j >= 4, j - 4 < NWIN))
            def _():
                pltpu.make_async_copy(bufs.at[b], acc_sh.at[dst_v.at[j - 4]], ssem).wait()

            @pl.when(j < NWIN)
            def _():
                for q in range(8):
                    sl = pl.ds(q * 16, 16)
                    widx3[b, sl] = src_v[j, sl] * 16 + g_glob
                pltpu.async_copy(x16_ref.at[widx3.at[b]], bufs.at[b], sems.at[b])

            @pl.when(jnp.logical_and(j >= 3, j - 3 < NWIN))
            def _():
                jm = j - 3
                bm = jm & 3
                pltpu.make_async_copy(x16_ref.at[widx3.at[bm]], bufs.at[bm], sems.at[bm]).wait()

                def _rb(g, carry2):
                    wv = e4[p // 2, pl.ds(jm * 128 + g * 16, 16)]
                    for a in range(16):
                        wb = jnp.full((16,), wv[a], jnp.float32)
                        r = g * 16 + a
                        bufs[bm, r, pl.ds(0, 16)] = bufs[bm, r, pl.ds(0, 16)] * wb
                    return carry2
                lax.fori_loop(0, 8, _rb, 0)
                pltpu.async_copy(bufs.at[bm], acc_sh.at[dst_v.at[jm]], ssem, add=True)
            return carry
        lax.fori_loop(0, NWIN + 4, _pb, 0)
        plsc.subcore_barrier()

        pltpu.sync_copy(acc_sh.at[pl.ds(cb, 625)], out_ref.at[g_glob, pl.ds(cb, 625)])
        plsc.subcore_barrier()


def _sc_edge(src3, dst3, ats, atd, mrow, x16):
    mesh = plsc.VectorSubcoreMesh(core_axis_name="c", subcore_axis_name="s")
    fn = pl.kernel(
        _sc_edge_body,
        out_type=jax.ShapeDtypeStruct((16, N, 16), jnp.float32),
        mesh=mesh,
        compiler_params=pltpu.CompilerParams(needs_layout_passes=False, use_tc_tiling_on_sc=False),
        scratch_types=[
            pltpu.VMEM((NWIN, WIN), jnp.int32),
            pltpu.VMEM((NWIN, WIN), jnp.int32),
            pltpu.VMEM((NWIN, WIN), jnp.int32),
            pltpu.VMEM((NWIN, WIN), jnp.int32),
            pltpu.VMEM((4, ET), jnp.float32),
            pltpu.VMEM((SEGP,), jnp.float32),
            pltpu.VMEM((SEGP,), jnp.float32),
            pltpu.VMEM((640,), jnp.float32),
            pltpu.VMEM((WIN, 16), jnp.float32),
            pltpu.VMEM((4, WIN, 16), jnp.float32),
            pltpu.VMEM((8, WIN), jnp.int32),
            pltpu.VMEM((16,), jnp.float32),
            pltpu.VMEM_SHARED((SEG_ALL,), jnp.float32),
            pltpu.VMEM_SHARED((NP, 16), jnp.float32),
            pltpu.SemaphoreType.DMA,
            pltpu.SemaphoreType.DMA((8,)),
            pltpu.SemaphoreType.DMA,
        ],
    )
    return fn(src3, dst3, ats, atd, mrow, x16)


def _pad_edges(ei):
    src = jnp.concatenate([ei[0], jnp.zeros((EPAD,), ei.dtype)]).astype(jnp.int32)
    dst = jnp.concatenate([ei[1], jnp.full((EPAD,), N, ei.dtype)]).astype(jnp.int32)
    return src.reshape(NSUB, NWIN, WIN), dst.reshape(NSUB, NWIN, WIN)


def _han_conv(x_dict, epad_dict, p):
    xn = {}
    out_lists = {t: [] for t in NODE_TYPES}
    for t in NODE_TYPES:
        W, b = p["proj"][t]
        xn[t] = _proj(x_dict[t], W, b)
    for et in EDGE_TYPES:
        src_t, _, dst_t = et
        k = "__".join(et)
        src3, dst3 = epad_dict[et]
        xs = xn[src_t]
        xd = xn[dst_t]
        ats = (xs.reshape(N, H, D) * p["att_src"][k]).sum(-1).T
        atd = (xd.reshape(N, H, D) * p["att_dst"][k]).sum(-1).T
        m = jnp.max(ats, axis=1) + jnp.max(atd, axis=1)
        m = jnp.where(m >= 0.0, m, m * 0.2)
        mrow = jnp.broadcast_to(m[:, None], (H, 16)).reshape(H * 16)
        ats = ats.reshape(H * N)
        atd = atd.reshape(H * N)
        x16 = xs.reshape(16 * N, 16)
        out3 = _sc_edge(src3, dst3, ats, atd, mrow, x16)
        out_lists[dst_t].append(out3.transpose(1, 0, 2).reshape(N, C))
    new_x = {}
    for t in NODE_TYPES:
        new_x[t] = _semantic(out_lists[t], p["k_W"], p["k_b"], p["q"])
    return new_x


def kernel(x_author, x_paper, edge_index_writes, edge_index_rev_writes, edge_index_cites, params):
    x_dict = {"author": x_author, "paper": x_paper}
    epad_dict = {
        EDGE_TYPES[0]: _pad_edges(edge_index_writes),
        EDGE_TYPES[1]: _pad_edges(edge_index_rev_writes),
        EDGE_TYPES[2]: _pad_edges(edge_index_cites),
    }
    for lp in params:
        x_dict = _han_conv(x_dict, epad_dict, lp)
    return (x_dict["author"], x_dict["paper"])
